# baseline (speedup 1.0000x reference)
import functools

import jax
import jax.numpy as jnp
from jax import lax
from jax.experimental import pallas as pl
from jax.experimental.pallas import tpu as pltpu
from jax.experimental.pallas import tpu_sc as plsc

N_USERS = 50000
N_TOTAL = 100000
EMB = 32
NNZ = 1600000
BATCH = 4096

NC = 2
NS = 16
L = 16

HALF = N_TOTAL // NC
ROWS_PER_TILE = 3128
ACC_ROWS = ROWS_PER_TILE * NS
LAST_ROWS = HALF - 15 * ROWS_PER_TILE

PSLICE = NNZ // (NC * NS)
SEG = 26000
CAPC = SEG * NC * NS
PCHUNK = 10000
PGROUPS = PCHUNK // L

EDGES_PER_TILE = CAPC // NS
E_CHUNK = 400
N_CHUNKS = EDGES_PER_TILE // E_CHUNK
GROUPS = E_CHUNK // L

B_PER_W = BATCH // (NC * NS)

_mesh = plsc.VectorSubcoreMesh(core_axis_name="c", subcore_axis_name="s")


_GATHER_DNUMS = lax.GatherDimensionNumbers(
    offset_dims=(), collapsed_slice_dims=(0,), start_index_map=(0,))


def _bcast(vec, lane):
    idx = jnp.full((L, 1), lane, jnp.int32)
    return lax.gather(vec, idx, _GATHER_DNUMS, (1,),
                      mode=lax.GatherScatterMode.PROMISE_IN_BOUNDS)


def _partition_body(rows_h, cols_h, vals_h, prow_h, pcol_h, pval_h,
                    inr, inc, inv, srow, scol, sval):
    c = lax.axis_index("c")
    s = lax.axis_index("s")
    wid = s * NC + c
    ebase = wid * PSLICE

    izero = jnp.zeros((L,), jnp.int32)
    fzero = jnp.zeros((L,), jnp.float32)

    for r in range(NC):
        def _zb(g, _):
            srow[pl.ds(g * L, L)] = izero
            scol[pl.ds(g * L, L)] = izero
            sval[pl.ds(g * L, L)] = fzero
            return 0

        lax.fori_loop(0, SEG // L, _zb, 0)

        def _chunk(k, cur):
            eb = ebase + k * PCHUNK
            pltpu.sync_copy(rows_h.at[pl.ds(eb, PCHUNK)], inr)
            pltpu.sync_copy(cols_h.at[pl.ds(eb, PCHUNK)], inc)
            pltpu.sync_copy(vals_h.at[pl.ds(eb, PCHUNK)], inv)

            def _grp(g, cur):
                r16 = inr[pl.ds(g * L, L)]
                c16 = inc[pl.ds(g * L, L)]
                v16 = inv[pl.ds(g * L, L)]
                upper01 = 1 + ((r16 - HALF) >> 31)
                folded = r16 - upper01 * HALF
                mask = upper01 == r
                n = plsc.all_reduce_population_count(mask)
                plsc.store_compressed(srow.at[pl.ds(cur, L)], folded, mask=mask)
                plsc.store_compressed(scol.at[pl.ds(cur, L)], c16, mask=mask)
                plsc.store_compressed(sval.at[pl.ds(cur, L)], v16, mask=mask)
                return jnp.minimum(cur + n[0], SEG)

            return lax.fori_loop(0, PGROUPS, _grp, cur)

        cur_end = lax.fori_loop(0, PSLICE // PCHUNK, _chunk, 0)
        srow[pl.ds(cur_end, L)] = izero
        scol[pl.ds(cur_end, L)] = izero
        sval[pl.ds(cur_end, L)] = fzero
        seg_base = r * CAPC + wid * SEG
        pltpu.sync_copy(srow.at[pl.ds(0, SEG)], prow_h.at[pl.ds(seg_base, SEG)])
        pltpu.sync_copy(scol.at[pl.ds(0, SEG)], pcol_h.at[pl.ds(seg_base, SEG)])
        pltpu.sync_copy(sval.at[pl.ds(0, SEG)], pval_h.at[pl.ds(seg_base, SEG)])


_partition = pl.kernel(
    _partition_body,
    out_type=(jax.ShapeDtypeStruct((NC * CAPC,), jnp.int32),
              jax.ShapeDtypeStruct((NC * CAPC,), jnp.int32),
              jax.ShapeDtypeStruct((NC * CAPC,), jnp.float32)),
    mesh=_mesh,
    compiler_params=pltpu.CompilerParams(
        use_tc_tiling_on_sc=False, needs_layout_passes=False),
    scratch_types=(
        [pltpu.VMEM((PCHUNK,), jnp.int32)] * 2
        + [pltpu.VMEM((PCHUNK,), jnp.float32)]
        + [pltpu.VMEM((SEG + L,), jnp.int32)] * 2
        + [pltpu.VMEM((SEG + L,), jnp.float32)]
    ),
)


def _hop_body(agg, rows_h, cols_h, vals_h, out_h,
              accum, colv0, colv1, rowv0, rowv1, valv0, valv1,
              idxv0, idxv1, gath0, gath1,
              sem_st0, sem_st1, sem_g0, sem_g1, sem_sc0, sem_sc1):
    c = lax.axis_index("c")
    s = lax.axis_index("s")
    colv = (colv0, colv1)
    rowv = (rowv0, rowv1)
    valv = (valv0, valv1)
    idxv = (idxv0, idxv1)
    gath = (gath0, gath1)
    sem_st = (sem_st0, sem_st1)
    sem_g = (sem_g0, sem_g1)
    sem_sc = (sem_sc0, sem_sc1)

    zero = jnp.zeros((L,), jnp.float32)

    def _zb(g, _):
        gath0[g, pl.ds(0, L)] = zero
        gath0[g, pl.ds(L, L)] = zero
        return 0

    lax.fori_loop(0, E_CHUNK, _zb, 0)
    zoff = 0
    while zoff < ROWS_PER_TILE:
        zn = min(E_CHUNK, ROWS_PER_TILE - zoff)
        pltpu.sync_copy(gath0.at[pl.ds(0, zn)],
                        accum.at[pl.ds(s * ROWS_PER_TILE + zoff, zn)])
        zoff += zn
    plsc.subcore_barrier()

    ebase = c * CAPC + s * EDGES_PER_TILE

    def _stage(k, b):
        eb = ebase + k * E_CHUNK
        pltpu.async_copy(cols_h.at[pl.ds(eb, E_CHUNK)], colv[b], sem_st[b])
        pltpu.async_copy(rows_h.at[pl.ds(eb, E_CHUNK)], rowv[b], sem_st[b])
        pltpu.async_copy(vals_h.at[pl.ds(eb, E_CHUNK)], valv[b], sem_st[b])

    def _stage_wait(k, b):
        eb = ebase + k * E_CHUNK
        pltpu.make_async_copy(cols_h.at[pl.ds(eb, E_CHUNK)], colv[b], sem_st[b]).wait()
        pltpu.make_async_copy(rows_h.at[pl.ds(eb, E_CHUNK)], rowv[b], sem_st[b]).wait()
        pltpu.make_async_copy(vals_h.at[pl.ds(eb, E_CHUNK)], valv[b], sem_st[b]).wait()

    _stage(0, 0)

    def _outer(g, _):
        for b in range(2):
            k = 2 * g + b

            @pl.when(g >= 1)
            def _wait_sc():
                pltpu.make_async_copy(gath[b], accum.at[idxv[b]], sem_sc[b]).wait()

            _stage_wait(k, b)
            pltpu.async_copy(agg.at[colv[b]], gath[b], sem_g[b])

            @pl.when(k + 1 < N_CHUNKS)
            def _prefetch():
                _stage(k + 1, 1 - b)

            pltpu.make_async_copy(agg.at[colv[b]], gath[b], sem_g[b]).wait()

            def _grp(gi, _):
                v16 = valv[b][pl.ds(gi * L, L)]
                idxv[b][pl.ds(gi * L, L)] = rowv[b][pl.ds(gi * L, L)]
                e0 = gi * L
                for e in range(L):
                    w = _bcast(v16, e)
                    gath[b][e0 + e, pl.ds(0, L)] = gath[b][e0 + e, pl.ds(0, L)] * w
                    gath[b][e0 + e, pl.ds(L, L)] = gath[b][e0 + e, pl.ds(L, L)] * w
                return 0

            lax.fori_loop(0, GROUPS, _grp, 0)
            pltpu.async_copy(gath[b], accum.at[idxv[b]], sem_sc[b], add=True)
        return 0

    lax.fori_loop(0, N_CHUNKS // 2, _outer, 0)
    for b in range(2):
        pltpu.make_async_copy(gath[b], accum.at[idxv[b]], sem_sc[b]).wait()
    plsc.subcore_barrier()

    @pl.when(s < NS - 1)
    def _flush_full():
        pltpu.sync_copy(
            accum.at[pl.ds(s * ROWS_PER_TILE, ROWS_PER_TILE)],
            out_h.at[pl.ds(c * HALF + s * ROWS_PER_TILE, ROWS_PER_TILE)])

    @pl.when(s == NS - 1)
    def _flush_last():
        pltpu.sync_copy(
            accum.at[pl.ds((NS - 1) * ROWS_PER_TILE, LAST_ROWS)],
            out_h.at[pl.ds(c * HALF + (NS - 1) * ROWS_PER_TILE, LAST_ROWS)])


_hop = pl.kernel(
    _hop_body,
    out_type=jax.ShapeDtypeStruct((N_TOTAL, EMB), jnp.float32),
    mesh=_mesh,
    compiler_params=pltpu.CompilerParams(use_tc_tiling_on_sc=False),
    scratch_types=(
        [pltpu.VMEM_SHARED((ACC_ROWS, EMB), jnp.float32)]
        + [pltpu.VMEM((E_CHUNK,), jnp.int32)] * 4
        + [pltpu.VMEM((E_CHUNK,), jnp.float32)] * 2
        + [pltpu.VMEM((E_CHUNK,), jnp.int32)] * 2
        + [pltpu.VMEM((E_CHUNK, EMB), jnp.float32)] * 2
        + [pltpu.SemaphoreType.DMA] * 6
    ),
)


def _final_body(e0, e1, e2, e3, users_h, pos_h, out_h,
                uidx, pidx, ub0, ub1, ub2, ub3, ib0, ib1, ib2, ib3, outv):
    c = lax.axis_index("c")
    s = lax.axis_index("s")
    wid = s * NC + c
    base = wid * B_PER_W

    pltpu.sync_copy(users_h.at[pl.ds(base, B_PER_W)], uidx)
    pltpu.sync_copy(pos_h.at[pl.ds(base, B_PER_W)], pidx)

    def _shift(g, _):
        pidx[pl.ds(g * L, L)] = pidx[pl.ds(g * L, L)] + N_USERS
        return 0

    lax.fori_loop(0, B_PER_W // L, _shift, 0)

    pltpu.sync_copy(e0.at[uidx], ub0)
    pltpu.sync_copy(e1.at[uidx], ub1)
    pltpu.sync_copy(e2.at[uidx], ub2)
    pltpu.sync_copy(e3.at[uidx], ub3)
    pltpu.sync_copy(e0.at[pidx], ib0)
    pltpu.sync_copy(e1.at[pidx], ib1)
    pltpu.sync_copy(e2.at[pidx], ib2)
    pltpu.sync_copy(e3.at[pidx], ib3)

    def _dot(b, _):
        u0 = (ub0[b, pl.ds(0, L)] + ub1[b, pl.ds(0, L)]
              + ub2[b, pl.ds(0, L)] + ub3[b, pl.ds(0, L)])
        u1 = (ub0[b, pl.ds(L, L)] + ub1[b, pl.ds(L, L)]
              + ub2[b, pl.ds(L, L)] + ub3[b, pl.ds(L, L)])
        i0 = (ib0[b, pl.ds(0, L)] + ib1[b, pl.ds(0, L)]
              + ib2[b, pl.ds(0, L)] + ib3[b, pl.ds(0, L)])
        i1 = (ib0[b, pl.ds(L, L)] + ib1[b, pl.ds(L, L)]
              + ib2[b, pl.ds(L, L)] + ib3[b, pl.ds(L, L)])
        p = (u0 * i0 + u1 * i1) * (1.0 / 16.0)
        csum = plsc.cumsum(p)
        lane = lax.broadcasted_iota(jnp.int32, (L,), 0)
        plsc.store_scatter(outv, [jnp.full((L,), b, jnp.int32)], csum,
                           mask=lane == L - 1)
        return 0

    lax.fori_loop(0, B_PER_W, _dot, 0)
    pltpu.sync_copy(outv, out_h.at[pl.ds(base, B_PER_W)])


_final = pl.kernel(
    _final_body,
    out_type=jax.ShapeDtypeStruct((BATCH,), jnp.float32),
    mesh=_mesh,
    compiler_params=pltpu.CompilerParams(
        use_tc_tiling_on_sc=False, needs_layout_passes=False),
    scratch_types=(
        [pltpu.VMEM((B_PER_W,), jnp.int32)] * 2
        + [pltpu.VMEM((B_PER_W, EMB), jnp.float32)] * 8
        + [pltpu.VMEM((B_PER_W,), jnp.float32)]
    ),
)


def kernel(user_embed, item_embed, adj_values, adj_indices, users, pos_items):
    all_embed = jnp.concatenate([user_embed, item_embed], axis=0)
    rows = adj_indices[0]
    cols = adj_indices[1]
    prow, pcol, pval = _partition(rows, cols, adj_values)
    e1 = _hop(all_embed, prow, pcol, pval)
    e2 = _hop(e1, prow, pcol, pval)
    e3 = _hop(e2, prow, pcol, pval)
    return _final(all_embed, e1, e2, e3, users, pos_items)


# speedup vs baseline: 2.7361x; 2.7361x over previous
import functools

import jax
import jax.numpy as jnp
from jax import lax
from jax.experimental import pallas as pl
from jax.experimental.pallas import tpu as pltpu
from jax.experimental.pallas import tpu_sc as plsc

N_USERS = 50000
N_TOTAL = 100000
EMB = 32
NNZ = 1600000
BATCH = 4096

NC = 2
NS = 16
L = 16

HALF = N_TOTAL // NC
ROWS_PER_TILE = 3128
ACC_ROWS = ROWS_PER_TILE * NS
LAST_ROWS = HALF - 15 * ROWS_PER_TILE

PSLICE = NNZ // (NC * NS)
SEG = 26000
CAPC = SEG * NC * NS
PCHUNK = 10000
PGROUPS = PCHUNK // L

EDGES_PER_TILE = CAPC // NS
E_CHUNK = 400
N_CHUNKS = EDGES_PER_TILE // E_CHUNK
GROUPS = E_CHUNK // L

B_PER_W = BATCH // (NC * NS)

_mesh = plsc.VectorSubcoreMesh(core_axis_name="c", subcore_axis_name="s")


_GATHER_DNUMS = lax.GatherDimensionNumbers(
    offset_dims=(), collapsed_slice_dims=(0,), start_index_map=(0,))


def _bcast(vec, lane):
    idx = jnp.full((L, 1), lane, jnp.int32)
    return lax.gather(vec, idx, _GATHER_DNUMS, (1,),
                      mode=lax.GatherScatterMode.PROMISE_IN_BOUNDS)


def _partition_body(rows_h, cols_h, vals_h, prow_h, pcol_h, pval_h,
                    inr, inc, inv, srow, scol, sval):
    c = lax.axis_index("c")
    s = lax.axis_index("s")
    wid = s * NC + c
    ebase = wid * PSLICE

    izero = jnp.zeros((L,), jnp.int32)
    fzero = jnp.zeros((L,), jnp.float32)

    lane = lax.broadcasted_iota(jnp.int32, (L,), 0)

    for r in range(NC):
        def _zb(g, _):
            pad = g * L + lane
            srow[pl.ds(g * L, L)] = pad
            scol[pl.ds(g * L, L)] = pad
            sval[pl.ds(g * L, L)] = fzero
            return 0

        lax.fori_loop(0, SEG // L, _zb, 0)

        def _chunk(k, cur):
            eb = ebase + k * PCHUNK
            pltpu.sync_copy(rows_h.at[pl.ds(eb, PCHUNK)], inr)
            pltpu.sync_copy(cols_h.at[pl.ds(eb, PCHUNK)], inc)
            pltpu.sync_copy(vals_h.at[pl.ds(eb, PCHUNK)], inv)

            def _grp(g, cur):
                r16 = inr[pl.ds(g * L, L)]
                c16 = inc[pl.ds(g * L, L)]
                v16 = inv[pl.ds(g * L, L)]
                upper01 = 1 + ((r16 - HALF) >> 31)
                folded = r16 - upper01 * HALF
                mask = upper01 == r
                n = plsc.all_reduce_population_count(mask)
                plsc.store_compressed(srow.at[pl.ds(cur, L)], folded, mask=mask)
                plsc.store_compressed(scol.at[pl.ds(cur, L)], c16, mask=mask)
                plsc.store_compressed(sval.at[pl.ds(cur, L)], v16, mask=mask)
                return jnp.minimum(cur + n[0], SEG)

            return lax.fori_loop(0, PGROUPS, _grp, cur)

        cur_end = lax.fori_loop(0, PSLICE // PCHUNK, _chunk, 0)
        srow[pl.ds(cur_end, L)] = cur_end + lane
        scol[pl.ds(cur_end, L)] = cur_end + lane
        sval[pl.ds(cur_end, L)] = fzero
        seg_base = r * CAPC + wid * SEG
        pltpu.sync_copy(srow.at[pl.ds(0, SEG)], prow_h.at[pl.ds(seg_base, SEG)])
        pltpu.sync_copy(scol.at[pl.ds(0, SEG)], pcol_h.at[pl.ds(seg_base, SEG)])
        pltpu.sync_copy(sval.at[pl.ds(0, SEG)], pval_h.at[pl.ds(seg_base, SEG)])


_partition = pl.kernel(
    _partition_body,
    out_type=(jax.ShapeDtypeStruct((NC * CAPC,), jnp.int32),
              jax.ShapeDtypeStruct((NC * CAPC,), jnp.int32),
              jax.ShapeDtypeStruct((NC * CAPC,), jnp.float32)),
    mesh=_mesh,
    compiler_params=pltpu.CompilerParams(
        use_tc_tiling_on_sc=False, needs_layout_passes=False),
    scratch_types=(
        [pltpu.VMEM((PCHUNK,), jnp.int32)] * 2
        + [pltpu.VMEM((PCHUNK,), jnp.float32)]
        + [pltpu.VMEM((SEG + L,), jnp.int32)] * 2
        + [pltpu.VMEM((SEG + L,), jnp.float32)]
    ),
)


def _hop_body(agg, rows_h, cols_h, vals_h, out_h,
              accum, colv0, colv1, rowv0, rowv1, valv0, valv1,
              idxv0, idxv1, gath0, gath1,
              sem_st0, sem_st1, sem_g0, sem_g1, sem_sc0, sem_sc1):
    c = lax.axis_index("c")
    s = lax.axis_index("s")
    colv = (colv0, colv1)
    rowv = (rowv0, rowv1)
    valv = (valv0, valv1)
    idxv = (idxv0, idxv1)
    gath = (gath0, gath1)
    sem_st = (sem_st0, sem_st1)
    sem_g = (sem_g0, sem_g1)
    sem_sc = (sem_sc0, sem_sc1)

    zero = jnp.zeros((L,), jnp.float32)

    def _zb(g, _):
        gath0[g, pl.ds(0, L)] = zero
        gath0[g, pl.ds(L, L)] = zero
        return 0

    lax.fori_loop(0, E_CHUNK, _zb, 0)
    zoff = 0
    while zoff < ROWS_PER_TILE:
        zn = min(E_CHUNK, ROWS_PER_TILE - zoff)
        pltpu.sync_copy(gath0.at[pl.ds(0, zn)],
                        accum.at[pl.ds(s * ROWS_PER_TILE + zoff, zn)])
        zoff += zn
    plsc.subcore_barrier()

    ebase = c * CAPC + s * EDGES_PER_TILE

    def _stage(k, b):
        eb = ebase + k * E_CHUNK
        pltpu.async_copy(cols_h.at[pl.ds(eb, E_CHUNK)], colv[b], sem_st[b])
        pltpu.async_copy(rows_h.at[pl.ds(eb, E_CHUNK)], rowv[b], sem_st[b])
        pltpu.async_copy(vals_h.at[pl.ds(eb, E_CHUNK)], valv[b], sem_st[b])

    def _stage_wait(k, b):
        eb = ebase + k * E_CHUNK
        pltpu.make_async_copy(cols_h.at[pl.ds(eb, E_CHUNK)], colv[b], sem_st[b]).wait()
        pltpu.make_async_copy(rows_h.at[pl.ds(eb, E_CHUNK)], rowv[b], sem_st[b]).wait()
        pltpu.make_async_copy(vals_h.at[pl.ds(eb, E_CHUNK)], valv[b], sem_st[b]).wait()

    _stage(0, 0)

    def _outer(g, _):
        for b in range(2):
            k = 2 * g + b

            @pl.when(g >= 1)
            def _wait_sc():
                pltpu.make_async_copy(gath[b], accum.at[idxv[b]], sem_sc[b]).wait()

            _stage_wait(k, b)
            pltpu.async_copy(agg.at[colv[b]], gath[b], sem_g[b])

            @pl.when(k + 1 < N_CHUNKS)
            def _prefetch():
                _stage(k + 1, 1 - b)

            pltpu.make_async_copy(agg.at[colv[b]], gath[b], sem_g[b]).wait()

            def _grp(gi, _):
                v16 = valv[b][pl.ds(gi * L, L)]
                idxv[b][pl.ds(gi * L, L)] = rowv[b][pl.ds(gi * L, L)]
                e0 = gi * L
                for e in range(L):
                    w = _bcast(v16, e)
                    gath[b][e0 + e, pl.ds(0, L)] = gath[b][e0 + e, pl.ds(0, L)] * w
                    gath[b][e0 + e, pl.ds(L, L)] = gath[b][e0 + e, pl.ds(L, L)] * w
                return 0

            lax.fori_loop(0, GROUPS, _grp, 0)
            pltpu.async_copy(gath[b], accum.at[idxv[b]], sem_sc[b], add=True)
        return 0

    lax.fori_loop(0, N_CHUNKS // 2, _outer, 0)
    for b in range(2):
        pltpu.make_async_copy(gath[b], accum.at[idxv[b]], sem_sc[b]).wait()
    plsc.subcore_barrier()

    @pl.when(s < NS - 1)
    def _flush_full():
        pltpu.sync_copy(
            accum.at[pl.ds(s * ROWS_PER_TILE, ROWS_PER_TILE)],
            out_h.at[pl.ds(c * HALF + s * ROWS_PER_TILE, ROWS_PER_TILE)])

    @pl.when(s == NS - 1)
    def _flush_last():
        pltpu.sync_copy(
            accum.at[pl.ds((NS - 1) * ROWS_PER_TILE, LAST_ROWS)],
            out_h.at[pl.ds(c * HALF + (NS - 1) * ROWS_PER_TILE, LAST_ROWS)])


_hop = pl.kernel(
    _hop_body,
    out_type=jax.ShapeDtypeStruct((N_TOTAL, EMB), jnp.float32),
    mesh=_mesh,
    compiler_params=pltpu.CompilerParams(use_tc_tiling_on_sc=False),
    scratch_types=(
        [pltpu.VMEM_SHARED((ACC_ROWS, EMB), jnp.float32)]
        + [pltpu.VMEM((E_CHUNK,), jnp.int32)] * 4
        + [pltpu.VMEM((E_CHUNK,), jnp.float32)] * 2
        + [pltpu.VMEM((E_CHUNK,), jnp.int32)] * 2
        + [pltpu.VMEM((E_CHUNK, EMB), jnp.float32)] * 2
        + [pltpu.SemaphoreType.DMA] * 6
    ),
)


def _final_body(e0, e1, e2, e3, users_h, pos_h, out_h,
                uidx, pidx, ub0, ub1, ub2, ub3, ib0, ib1, ib2, ib3, outv):
    c = lax.axis_index("c")
    s = lax.axis_index("s")
    wid = s * NC + c
    base = wid * B_PER_W

    pltpu.sync_copy(users_h.at[pl.ds(base, B_PER_W)], uidx)
    pltpu.sync_copy(pos_h.at[pl.ds(base, B_PER_W)], pidx)

    def _shift(g, _):
        pidx[pl.ds(g * L, L)] = pidx[pl.ds(g * L, L)] + N_USERS
        return 0

    lax.fori_loop(0, B_PER_W // L, _shift, 0)

    pltpu.sync_copy(e0.at[uidx], ub0)
    pltpu.sync_copy(e1.at[uidx], ub1)
    pltpu.sync_copy(e2.at[uidx], ub2)
    pltpu.sync_copy(e3.at[uidx], ub3)
    pltpu.sync_copy(e0.at[pidx], ib0)
    pltpu.sync_copy(e1.at[pidx], ib1)
    pltpu.sync_copy(e2.at[pidx], ib2)
    pltpu.sync_copy(e3.at[pidx], ib3)

    def _dot(b, _):
        u0 = (ub0[b, pl.ds(0, L)] + ub1[b, pl.ds(0, L)]
              + ub2[b, pl.ds(0, L)] + ub3[b, pl.ds(0, L)])
        u1 = (ub0[b, pl.ds(L, L)] + ub1[b, pl.ds(L, L)]
              + ub2[b, pl.ds(L, L)] + ub3[b, pl.ds(L, L)])
        i0 = (ib0[b, pl.ds(0, L)] + ib1[b, pl.ds(0, L)]
              + ib2[b, pl.ds(0, L)] + ib3[b, pl.ds(0, L)])
        i1 = (ib0[b, pl.ds(L, L)] + ib1[b, pl.ds(L, L)]
              + ib2[b, pl.ds(L, L)] + ib3[b, pl.ds(L, L)])
        p = (u0 * i0 + u1 * i1) * (1.0 / 16.0)
        csum = plsc.cumsum(p)
        lane = lax.broadcasted_iota(jnp.int32, (L,), 0)
        plsc.store_scatter(outv, [jnp.full((L,), b, jnp.int32)], csum,
                           mask=lane == L - 1)
        return 0

    lax.fori_loop(0, B_PER_W, _dot, 0)
    pltpu.sync_copy(outv, out_h.at[pl.ds(base, B_PER_W)])


_final = pl.kernel(
    _final_body,
    out_type=jax.ShapeDtypeStruct((BATCH,), jnp.float32),
    mesh=_mesh,
    compiler_params=pltpu.CompilerParams(
        use_tc_tiling_on_sc=False, needs_layout_passes=False),
    scratch_types=(
        [pltpu.VMEM((B_PER_W,), jnp.int32)] * 2
        + [pltpu.VMEM((B_PER_W, EMB), jnp.float32)] * 8
        + [pltpu.VMEM((B_PER_W,), jnp.float32)]
    ),
)


def kernel(user_embed, item_embed, adj_values, adj_indices, users, pos_items):
    all_embed = jnp.concatenate([user_embed, item_embed], axis=0)
    rows = adj_indices[0]
    cols = adj_indices[1]
    prow, pcol, pval = _partition(rows, cols, adj_values)
    e1 = _hop(all_embed, prow, pcol, pval)
    e2 = _hop(e1, prow, pcol, pval)
    e3 = _hop(e2, prow, pcol, pval)
    return _final(all_embed, e1, e2, e3, users, pos_items)


# speedup vs baseline: 3.4640x; 1.2660x over previous
import functools

import jax
import jax.numpy as jnp
from jax import lax
from jax.experimental import pallas as pl
from jax.experimental.pallas import tpu as pltpu
from jax.experimental.pallas import tpu_sc as plsc

N_USERS = 50000
N_TOTAL = 100000
EMB = 32
NNZ = 1600000
BATCH = 4096

NC = 2
NS = 16
L = 16

HALF = N_TOTAL // NC
ROWS_PER_TILE = 3128
ACC_ROWS = ROWS_PER_TILE * NS
LAST_ROWS = HALF - 15 * ROWS_PER_TILE

PSLICE = NNZ // (NC * NS)
SEG = 26000
CAPC = SEG * NC * NS
PCHUNK = 10000
PGROUPS = PCHUNK // L

EDGES_PER_TILE = CAPC // NS
E_CHUNK = 400
N_CHUNKS = EDGES_PER_TILE // E_CHUNK
GROUPS = E_CHUNK // L

B_PER_W = BATCH // (NC * NS)

_mesh = plsc.VectorSubcoreMesh(core_axis_name="c", subcore_axis_name="s")


_GATHER_DNUMS = lax.GatherDimensionNumbers(
    offset_dims=(), collapsed_slice_dims=(0,), start_index_map=(0,))


def _bcast(vec, lane):
    idx = jnp.full((L, 1), lane, jnp.int32)
    return lax.gather(vec, idx, _GATHER_DNUMS, (1,),
                      mode=lax.GatherScatterMode.PROMISE_IN_BOUNDS)


def _partition_body(rows_h, cols_h, vals_h, prow_h, pcol_h, pval_h,
                    inr, inc, inv, srow, scol, sval):
    c = lax.axis_index("c")
    s = lax.axis_index("s")
    wid = s * NC + c
    ebase = wid * PSLICE

    izero = jnp.zeros((L,), jnp.int32)
    fzero = jnp.zeros((L,), jnp.float32)

    lane = lax.broadcasted_iota(jnp.int32, (L,), 0)

    for r in range(NC):
        def _zb(g, _):
            pad = g * L + lane
            srow[pl.ds(g * L, L)] = pad
            scol[pl.ds(g * L, L)] = pad
            sval[pl.ds(g * L, L)] = fzero
            return 0

        lax.fori_loop(0, SEG // L, _zb, 0)

        def _chunk(k, cur):
            eb = ebase + k * PCHUNK
            pltpu.sync_copy(rows_h.at[pl.ds(eb, PCHUNK)], inr)
            pltpu.sync_copy(cols_h.at[pl.ds(eb, PCHUNK)], inc)
            pltpu.sync_copy(vals_h.at[pl.ds(eb, PCHUNK)], inv)

            def _grp(g, cur):
                r16 = inr[pl.ds(g * L, L)]
                c16 = inc[pl.ds(g * L, L)]
                v16 = inv[pl.ds(g * L, L)]
                upper01 = 1 + ((r16 - HALF) >> 31)
                folded = r16 - upper01 * HALF
                mask = upper01 == r
                n = plsc.all_reduce_population_count(mask)
                plsc.store_compressed(srow.at[pl.ds(cur, L)], folded, mask=mask)
                plsc.store_compressed(scol.at[pl.ds(cur, L)], c16, mask=mask)
                plsc.store_compressed(sval.at[pl.ds(cur, L)], v16, mask=mask)
                return jnp.minimum(cur + n[0], SEG)

            return lax.fori_loop(0, PGROUPS, _grp, cur)

        cur_end = lax.fori_loop(0, PSLICE // PCHUNK, _chunk, 0)
        srow[pl.ds(cur_end, L)] = cur_end + lane
        scol[pl.ds(cur_end, L)] = cur_end + lane
        sval[pl.ds(cur_end, L)] = fzero
        seg_base = r * CAPC + wid * SEG
        pltpu.sync_copy(srow.at[pl.ds(0, SEG)], prow_h.at[pl.ds(seg_base, SEG)])
        pltpu.sync_copy(scol.at[pl.ds(0, SEG)], pcol_h.at[pl.ds(seg_base, SEG)])
        pltpu.sync_copy(sval.at[pl.ds(0, SEG)], pval_h.at[pl.ds(seg_base, SEG)])


_partition = pl.kernel(
    _partition_body,
    out_type=(jax.ShapeDtypeStruct((NC * CAPC,), jnp.int32),
              jax.ShapeDtypeStruct((NC * CAPC,), jnp.int32),
              jax.ShapeDtypeStruct((NC * CAPC,), jnp.float32)),
    mesh=_mesh,
    compiler_params=pltpu.CompilerParams(
        use_tc_tiling_on_sc=False, needs_layout_passes=False),
    scratch_types=(
        [pltpu.VMEM((PCHUNK,), jnp.int32)] * 2
        + [pltpu.VMEM((PCHUNK,), jnp.float32)]
        + [pltpu.VMEM((SEG + L,), jnp.int32)] * 2
        + [pltpu.VMEM((SEG + L,), jnp.float32)]
    ),
)


def _hop_body(agg, rows_h, cols_h, vals_h, out_h,
              accum, colv0, colv1, rowv0, rowv1, valv0, valv1,
              idxv0, idxv1, gath0, gath1,
              sem_st0, sem_st1, sem_g0, sem_g1, sem_sc0, sem_sc1):
    c = lax.axis_index("c")
    s = lax.axis_index("s")
    colv = (colv0, colv1)
    rowv = (rowv0, rowv1)
    valv = (valv0, valv1)
    idxv = (idxv0, idxv1)
    gath = (gath0, gath1)
    sem_st = (sem_st0, sem_st1)
    sem_g = (sem_g0, sem_g1)
    sem_sc = (sem_sc0, sem_sc1)

    zero = jnp.zeros((L,), jnp.float32)

    def _zb(g, _):
        gath0[g, pl.ds(0, L)] = zero
        gath0[g, pl.ds(L, L)] = zero
        return 0

    lax.fori_loop(0, E_CHUNK, _zb, 0)
    zoff = 0
    while zoff < ROWS_PER_TILE:
        zn = min(E_CHUNK, ROWS_PER_TILE - zoff)
        pltpu.sync_copy(gath0.at[pl.ds(0, zn)],
                        accum.at[pl.ds(s * ROWS_PER_TILE + zoff, zn)])
        zoff += zn
    plsc.subcore_barrier()

    ebase = c * CAPC + s * EDGES_PER_TILE

    def _stage(k, b):
        eb = ebase + k * E_CHUNK
        pltpu.async_copy(cols_h.at[pl.ds(eb, E_CHUNK)], colv[b], sem_st[b])
        pltpu.async_copy(rows_h.at[pl.ds(eb, E_CHUNK)], rowv[b], sem_st[b])
        pltpu.async_copy(vals_h.at[pl.ds(eb, E_CHUNK)], valv[b], sem_st[b])

    def _stage_wait(k, b):
        eb = ebase + k * E_CHUNK
        pltpu.make_async_copy(cols_h.at[pl.ds(eb, E_CHUNK)], colv[b], sem_st[b]).wait()
        pltpu.make_async_copy(rows_h.at[pl.ds(eb, E_CHUNK)], rowv[b], sem_st[b]).wait()
        pltpu.make_async_copy(vals_h.at[pl.ds(eb, E_CHUNK)], valv[b], sem_st[b]).wait()

    _stage(0, 0)
    _stage_wait(0, 0)
    pltpu.async_copy(agg.at[colv[0]], gath[0], sem_g[0])
    _stage(1, 1)

    def _outer(g, _):
        for b in range(2):
            k = 2 * g + b
            nb = 1 - b

            @pl.when(k + 1 < N_CHUNKS)
            def _issue_next_gather():
                _stage_wait(k + 1, nb)

                @pl.when(k >= 1)
                def _wait_prev_scatter():
                    pltpu.make_async_copy(
                        gath[nb], accum.at[idxv[nb]], sem_sc[nb]).wait()

                pltpu.async_copy(agg.at[colv[nb]], gath[nb], sem_g[nb])

            pltpu.make_async_copy(agg.at[colv[b]], gath[b], sem_g[b]).wait()

            def _grp(gi, _):
                v16 = valv[b][pl.ds(gi * L, L)]
                idxv[b][pl.ds(gi * L, L)] = rowv[b][pl.ds(gi * L, L)]
                e0 = gi * L
                for e in range(L):
                    w = _bcast(v16, e)
                    gath[b][e0 + e, pl.ds(0, L)] = gath[b][e0 + e, pl.ds(0, L)] * w
                    gath[b][e0 + e, pl.ds(L, L)] = gath[b][e0 + e, pl.ds(L, L)] * w
                return 0

            lax.fori_loop(0, GROUPS, _grp, 0)

            @pl.when(k + 2 < N_CHUNKS)
            def _prefetch():
                _stage(k + 2, b)

            pltpu.async_copy(gath[b], accum.at[idxv[b]], sem_sc[b], add=True)
        return 0

    lax.fori_loop(0, N_CHUNKS // 2, _outer, 0)
    for b in range(2):
        pltpu.make_async_copy(gath[b], accum.at[idxv[b]], sem_sc[b]).wait()
    plsc.subcore_barrier()

    @pl.when(s < NS - 1)
    def _flush_full():
        pltpu.sync_copy(
            accum.at[pl.ds(s * ROWS_PER_TILE, ROWS_PER_TILE)],
            out_h.at[pl.ds(c * HALF + s * ROWS_PER_TILE, ROWS_PER_TILE)])

    @pl.when(s == NS - 1)
    def _flush_last():
        pltpu.sync_copy(
            accum.at[pl.ds((NS - 1) * ROWS_PER_TILE, LAST_ROWS)],
            out_h.at[pl.ds(c * HALF + (NS - 1) * ROWS_PER_TILE, LAST_ROWS)])


_hop = pl.kernel(
    _hop_body,
    out_type=jax.ShapeDtypeStruct((N_TOTAL, EMB), jnp.float32),
    mesh=_mesh,
    compiler_params=pltpu.CompilerParams(use_tc_tiling_on_sc=False),
    scratch_types=(
        [pltpu.VMEM_SHARED((ACC_ROWS, EMB), jnp.float32)]
        + [pltpu.VMEM((E_CHUNK,), jnp.int32)] * 4
        + [pltpu.VMEM((E_CHUNK,), jnp.float32)] * 2
        + [pltpu.VMEM((E_CHUNK,), jnp.int32)] * 2
        + [pltpu.VMEM((E_CHUNK, EMB), jnp.float32)] * 2
        + [pltpu.SemaphoreType.DMA] * 6
    ),
)


def _final_body(e0, e1, e2, e3, users_h, pos_h, out_h,
                uidx, pidx, ub0, ub1, ub2, ub3, ib0, ib1, ib2, ib3, outv):
    c = lax.axis_index("c")
    s = lax.axis_index("s")
    wid = s * NC + c
    base = wid * B_PER_W

    pltpu.sync_copy(users_h.at[pl.ds(base, B_PER_W)], uidx)
    pltpu.sync_copy(pos_h.at[pl.ds(base, B_PER_W)], pidx)

    def _shift(g, _):
        pidx[pl.ds(g * L, L)] = pidx[pl.ds(g * L, L)] + N_USERS
        return 0

    lax.fori_loop(0, B_PER_W // L, _shift, 0)

    pltpu.sync_copy(e0.at[uidx], ub0)
    pltpu.sync_copy(e1.at[uidx], ub1)
    pltpu.sync_copy(e2.at[uidx], ub2)
    pltpu.sync_copy(e3.at[uidx], ub3)
    pltpu.sync_copy(e0.at[pidx], ib0)
    pltpu.sync_copy(e1.at[pidx], ib1)
    pltpu.sync_copy(e2.at[pidx], ib2)
    pltpu.sync_copy(e3.at[pidx], ib3)

    def _dot(b, _):
        u0 = (ub0[b, pl.ds(0, L)] + ub1[b, pl.ds(0, L)]
              + ub2[b, pl.ds(0, L)] + ub3[b, pl.ds(0, L)])
        u1 = (ub0[b, pl.ds(L, L)] + ub1[b, pl.ds(L, L)]
              + ub2[b, pl.ds(L, L)] + ub3[b, pl.ds(L, L)])
        i0 = (ib0[b, pl.ds(0, L)] + ib1[b, pl.ds(0, L)]
              + ib2[b, pl.ds(0, L)] + ib3[b, pl.ds(0, L)])
        i1 = (ib0[b, pl.ds(L, L)] + ib1[b, pl.ds(L, L)]
              + ib2[b, pl.ds(L, L)] + ib3[b, pl.ds(L, L)])
        p = (u0 * i0 + u1 * i1) * (1.0 / 16.0)
        csum = plsc.cumsum(p)
        lane = lax.broadcasted_iota(jnp.int32, (L,), 0)
        plsc.store_scatter(outv, [jnp.full((L,), b, jnp.int32)], csum,
                           mask=lane == L - 1)
        return 0

    lax.fori_loop(0, B_PER_W, _dot, 0)
    pltpu.sync_copy(outv, out_h.at[pl.ds(base, B_PER_W)])


_final = pl.kernel(
    _final_body,
    out_type=jax.ShapeDtypeStruct((BATCH,), jnp.float32),
    mesh=_mesh,
    compiler_params=pltpu.CompilerParams(
        use_tc_tiling_on_sc=False, needs_layout_passes=False),
    scratch_types=(
        [pltpu.VMEM((B_PER_W,), jnp.int32)] * 2
        + [pltpu.VMEM((B_PER_W, EMB), jnp.float32)] * 8
        + [pltpu.VMEM((B_PER_W,), jnp.float32)]
    ),
)


def kernel(user_embed, item_embed, adj_values, adj_indices, users, pos_items):
    all_embed = jnp.concatenate([user_embed, item_embed], axis=0)
    rows = adj_indices[0]
    cols = adj_indices[1]
    prow, pcol, pval = _partition(rows, cols, adj_values)
    e1 = _hop(all_embed, prow, pcol, pval)
    e2 = _hop(e1, prow, pcol, pval)
    e3 = _hop(e2, prow, pcol, pval)
    return _final(all_embed, e1, e2, e3, users, pos_items)


# speedup vs baseline: 3.6038x; 1.0404x over previous
import functools

import jax
import jax.numpy as jnp
from jax import lax
from jax.experimental import pallas as pl
from jax.experimental.pallas import tpu as pltpu
from jax.experimental.pallas import tpu_sc as plsc

N_USERS = 50000
N_TOTAL = 100000
EMB = 32
NNZ = 1600000
BATCH = 4096

NC = 2
NS = 16
L = 16

HALF = N_TOTAL // NC
ROWS_PER_TILE = 3128
ACC_ROWS = ROWS_PER_TILE * NS
LAST_ROWS = HALF - 15 * ROWS_PER_TILE

PSLICE = NNZ // (NC * NS)
SEG = 26000
CAPC = SEG * NC * NS
PCHUNK = 2000
PGROUPS = PCHUNK // L

EDGES_PER_TILE = CAPC // NS
E_CHUNK = 400
N_CHUNKS = EDGES_PER_TILE // E_CHUNK
GROUPS = E_CHUNK // L

B_PER_W = BATCH // (NC * NS)

_mesh = plsc.VectorSubcoreMesh(core_axis_name="c", subcore_axis_name="s")


_GATHER_DNUMS = lax.GatherDimensionNumbers(
    offset_dims=(), collapsed_slice_dims=(0,), start_index_map=(0,))


def _bcast(vec, lane):
    idx = jnp.full((L, 1), lane, jnp.int32)
    return lax.gather(vec, idx, _GATHER_DNUMS, (1,),
                      mode=lax.GatherScatterMode.PROMISE_IN_BOUNDS)


def _partition_body(rows_h, cols_h, vals_h, prow_h, pcol_h, pval_h,
                    inr0, inr1, inc0, inc1, inv0, inv1, srow, scol, sval,
                    sem_p0, sem_p1):
    c = lax.axis_index("c")
    s = lax.axis_index("s")
    wid = s * NC + c
    ebase = wid * PSLICE
    inr = (inr0, inr1)
    inc = (inc0, inc1)
    inv = (inv0, inv1)
    sem_p = (sem_p0, sem_p1)

    izero = jnp.zeros((L,), jnp.int32)
    fzero = jnp.zeros((L,), jnp.float32)

    lane = lax.broadcasted_iota(jnp.int32, (L,), 0)

    def _pstage(k, b):
        eb = ebase + k * PCHUNK
        pltpu.async_copy(rows_h.at[pl.ds(eb, PCHUNK)], inr[b], sem_p[b])
        pltpu.async_copy(cols_h.at[pl.ds(eb, PCHUNK)], inc[b], sem_p[b])
        pltpu.async_copy(vals_h.at[pl.ds(eb, PCHUNK)], inv[b], sem_p[b])

    def _pstage_wait(k, b):
        eb = ebase + k * PCHUNK
        pltpu.make_async_copy(rows_h.at[pl.ds(eb, PCHUNK)], inr[b], sem_p[b]).wait()
        pltpu.make_async_copy(cols_h.at[pl.ds(eb, PCHUNK)], inc[b], sem_p[b]).wait()
        pltpu.make_async_copy(vals_h.at[pl.ds(eb, PCHUNK)], inv[b], sem_p[b]).wait()

    NPC = PSLICE // PCHUNK

    for r in range(NC):
        _pstage(0, 0)

        def _zb(g, _):
            pad = g * L + lane
            srow[pl.ds(g * L, L)] = pad
            scol[pl.ds(g * L, L)] = pad
            sval[pl.ds(g * L, L)] = fzero
            return 0

        lax.fori_loop(0, SEG // L, _zb, 0)

        def _outerp(gk, cur):
            for b in range(2):
                k = 2 * gk + b
                _pstage_wait(k, b)

                @pl.when(k + 1 < NPC)
                def _pf():
                    _pstage(k + 1, 1 - b)

                def _grp(g, cur):
                    r16 = inr[b][pl.ds(g * L, L)]
                    c16 = inc[b][pl.ds(g * L, L)]
                    v16 = inv[b][pl.ds(g * L, L)]
                    upper01 = 1 + ((r16 - HALF) >> 31)
                    folded = r16 - upper01 * HALF
                    mask = upper01 == r
                    n = plsc.all_reduce_population_count(mask)
                    plsc.store_compressed(srow.at[pl.ds(cur, L)], folded, mask=mask)
                    plsc.store_compressed(scol.at[pl.ds(cur, L)], c16, mask=mask)
                    plsc.store_compressed(sval.at[pl.ds(cur, L)], v16, mask=mask)
                    return jnp.minimum(cur + n[0], SEG)

                cur = lax.fori_loop(0, PGROUPS, _grp, cur)
            return cur

        cur_mid = lax.fori_loop(0, NPC // 2, _outerp, 0)

        _pstage_wait(NPC - 1, 0)

        def _grp_tail(g, cur):
            r16 = inr[0][pl.ds(g * L, L)]
            c16 = inc[0][pl.ds(g * L, L)]
            v16 = inv[0][pl.ds(g * L, L)]
            upper01 = 1 + ((r16 - HALF) >> 31)
            folded = r16 - upper01 * HALF
            mask = upper01 == r
            n = plsc.all_reduce_population_count(mask)
            plsc.store_compressed(srow.at[pl.ds(cur, L)], folded, mask=mask)
            plsc.store_compressed(scol.at[pl.ds(cur, L)], c16, mask=mask)
            plsc.store_compressed(sval.at[pl.ds(cur, L)], v16, mask=mask)
            return jnp.minimum(cur + n[0], SEG)

        cur_end = lax.fori_loop(0, PGROUPS, _grp_tail, cur_mid)
        srow[pl.ds(cur_end, L)] = cur_end + lane
        scol[pl.ds(cur_end, L)] = cur_end + lane
        sval[pl.ds(cur_end, L)] = fzero
        seg_base = r * CAPC + wid * SEG
        pltpu.sync_copy(srow.at[pl.ds(0, SEG)], prow_h.at[pl.ds(seg_base, SEG)])
        pltpu.sync_copy(scol.at[pl.ds(0, SEG)], pcol_h.at[pl.ds(seg_base, SEG)])
        pltpu.sync_copy(sval.at[pl.ds(0, SEG)], pval_h.at[pl.ds(seg_base, SEG)])


_partition = pl.kernel(
    _partition_body,
    out_type=(jax.ShapeDtypeStruct((NC * CAPC,), jnp.int32),
              jax.ShapeDtypeStruct((NC * CAPC,), jnp.int32),
              jax.ShapeDtypeStruct((NC * CAPC,), jnp.float32)),
    mesh=_mesh,
    compiler_params=pltpu.CompilerParams(
        use_tc_tiling_on_sc=False, needs_layout_passes=False),
    scratch_types=(
        [pltpu.VMEM((PCHUNK,), jnp.int32)] * 4
        + [pltpu.VMEM((PCHUNK,), jnp.float32)] * 2
        + [pltpu.VMEM((SEG + L,), jnp.int32)] * 2
        + [pltpu.VMEM((SEG + L,), jnp.float32)]
        + [pltpu.SemaphoreType.DMA] * 2
    ),
)


def _hop_body(agg, rows_h, cols_h, vals_h, out_h,
              accum, colv0, colv1, rowv0, rowv1, valv0, valv1,
              idxv0, idxv1, gath0, gath1,
              sem_st0, sem_st1, sem_g0, sem_g1, sem_sc0, sem_sc1):
    c = lax.axis_index("c")
    s = lax.axis_index("s")
    colv = (colv0, colv1)
    rowv = (rowv0, rowv1)
    valv = (valv0, valv1)
    idxv = (idxv0, idxv1)
    gath = (gath0, gath1)
    sem_st = (sem_st0, sem_st1)
    sem_g = (sem_g0, sem_g1)
    sem_sc = (sem_sc0, sem_sc1)

    zero = jnp.zeros((L,), jnp.float32)

    def _zb(g, _):
        gath0[g, pl.ds(0, L)] = zero
        gath0[g, pl.ds(L, L)] = zero
        return 0

    lax.fori_loop(0, E_CHUNK, _zb, 0)
    zoff = 0
    while zoff < ROWS_PER_TILE:
        zn = min(E_CHUNK, ROWS_PER_TILE - zoff)
        pltpu.sync_copy(gath0.at[pl.ds(0, zn)],
                        accum.at[pl.ds(s * ROWS_PER_TILE + zoff, zn)])
        zoff += zn
    plsc.subcore_barrier()

    ebase = c * CAPC + s * EDGES_PER_TILE

    def _stage(k, b):
        eb = ebase + k * E_CHUNK
        pltpu.async_copy(cols_h.at[pl.ds(eb, E_CHUNK)], colv[b], sem_st[b])
        pltpu.async_copy(rows_h.at[pl.ds(eb, E_CHUNK)], rowv[b], sem_st[b])
        pltpu.async_copy(vals_h.at[pl.ds(eb, E_CHUNK)], valv[b], sem_st[b])

    def _stage_wait(k, b):
        eb = ebase + k * E_CHUNK
        pltpu.make_async_copy(cols_h.at[pl.ds(eb, E_CHUNK)], colv[b], sem_st[b]).wait()
        pltpu.make_async_copy(rows_h.at[pl.ds(eb, E_CHUNK)], rowv[b], sem_st[b]).wait()
        pltpu.make_async_copy(vals_h.at[pl.ds(eb, E_CHUNK)], valv[b], sem_st[b]).wait()

    _stage(0, 0)
    _stage_wait(0, 0)
    pltpu.async_copy(agg.at[colv[0]], gath[0], sem_g[0])
    _stage(1, 1)

    def _outer(g, _):
        for b in range(2):
            k = 2 * g + b
            nb = 1 - b

            @pl.when(k + 1 < N_CHUNKS)
            def _issue_next_gather():
                _stage_wait(k + 1, nb)

                @pl.when(k >= 1)
                def _wait_prev_scatter():
                    pltpu.make_async_copy(
                        gath[nb], accum.at[idxv[nb]], sem_sc[nb]).wait()

                pltpu.async_copy(agg.at[colv[nb]], gath[nb], sem_g[nb])

            pltpu.make_async_copy(agg.at[colv[b]], gath[b], sem_g[b]).wait()

            def _grp(gi, _):
                v16 = valv[b][pl.ds(gi * L, L)]
                idxv[b][pl.ds(gi * L, L)] = rowv[b][pl.ds(gi * L, L)]
                e0 = gi * L
                for e in range(L):
                    w = _bcast(v16, e)
                    gath[b][e0 + e, pl.ds(0, L)] = gath[b][e0 + e, pl.ds(0, L)] * w
                    gath[b][e0 + e, pl.ds(L, L)] = gath[b][e0 + e, pl.ds(L, L)] * w
                return 0

            lax.fori_loop(0, GROUPS, _grp, 0)

            @pl.when(k + 2 < N_CHUNKS)
            def _prefetch():
                _stage(k + 2, b)

            pltpu.async_copy(gath[b], accum.at[idxv[b]], sem_sc[b], add=True)
        return 0

    lax.fori_loop(0, N_CHUNKS // 2, _outer, 0)
    for b in range(2):
        pltpu.make_async_copy(gath[b], accum.at[idxv[b]], sem_sc[b]).wait()
    plsc.subcore_barrier()

    @pl.when(s < NS - 1)
    def _flush_full():
        pltpu.sync_copy(
            accum.at[pl.ds(s * ROWS_PER_TILE, ROWS_PER_TILE)],
            out_h.at[pl.ds(c * HALF + s * ROWS_PER_TILE, ROWS_PER_TILE)])

    @pl.when(s == NS - 1)
    def _flush_last():
        pltpu.sync_copy(
            accum.at[pl.ds((NS - 1) * ROWS_PER_TILE, LAST_ROWS)],
            out_h.at[pl.ds(c * HALF + (NS - 1) * ROWS_PER_TILE, LAST_ROWS)])


_hop = pl.kernel(
    _hop_body,
    out_type=jax.ShapeDtypeStruct((N_TOTAL, EMB), jnp.float32),
    mesh=_mesh,
    compiler_params=pltpu.CompilerParams(use_tc_tiling_on_sc=False),
    scratch_types=(
        [pltpu.VMEM_SHARED((ACC_ROWS, EMB), jnp.float32)]
        + [pltpu.VMEM((E_CHUNK,), jnp.int32)] * 4
        + [pltpu.VMEM((E_CHUNK,), jnp.float32)] * 2
        + [pltpu.VMEM((E_CHUNK,), jnp.int32)] * 2
        + [pltpu.VMEM((E_CHUNK, EMB), jnp.float32)] * 2
        + [pltpu.SemaphoreType.DMA] * 6
    ),
)


def _final_body(e0, e1, e2, e3, users_h, pos_h, out_h,
                uidx, pidx, ub0, ub1, ub2, ub3, ib0, ib1, ib2, ib3, outv):
    c = lax.axis_index("c")
    s = lax.axis_index("s")
    wid = s * NC + c
    base = wid * B_PER_W

    pltpu.sync_copy(users_h.at[pl.ds(base, B_PER_W)], uidx)
    pltpu.sync_copy(pos_h.at[pl.ds(base, B_PER_W)], pidx)

    def _shift(g, _):
        pidx[pl.ds(g * L, L)] = pidx[pl.ds(g * L, L)] + N_USERS
        return 0

    lax.fori_loop(0, B_PER_W // L, _shift, 0)

    pltpu.sync_copy(e0.at[uidx], ub0)
    pltpu.sync_copy(e1.at[uidx], ub1)
    pltpu.sync_copy(e2.at[uidx], ub2)
    pltpu.sync_copy(e3.at[uidx], ub3)
    pltpu.sync_copy(e0.at[pidx], ib0)
    pltpu.sync_copy(e1.at[pidx], ib1)
    pltpu.sync_copy(e2.at[pidx], ib2)
    pltpu.sync_copy(e3.at[pidx], ib3)

    def _dot(b, _):
        u0 = (ub0[b, pl.ds(0, L)] + ub1[b, pl.ds(0, L)]
              + ub2[b, pl.ds(0, L)] + ub3[b, pl.ds(0, L)])
        u1 = (ub0[b, pl.ds(L, L)] + ub1[b, pl.ds(L, L)]
              + ub2[b, pl.ds(L, L)] + ub3[b, pl.ds(L, L)])
        i0 = (ib0[b, pl.ds(0, L)] + ib1[b, pl.ds(0, L)]
              + ib2[b, pl.ds(0, L)] + ib3[b, pl.ds(0, L)])
        i1 = (ib0[b, pl.ds(L, L)] + ib1[b, pl.ds(L, L)]
              + ib2[b, pl.ds(L, L)] + ib3[b, pl.ds(L, L)])
        p = (u0 * i0 + u1 * i1) * (1.0 / 16.0)
        csum = plsc.cumsum(p)
        lane = lax.broadcasted_iota(jnp.int32, (L,), 0)
        plsc.store_scatter(outv, [jnp.full((L,), b, jnp.int32)], csum,
                           mask=lane == L - 1)
        return 0

    lax.fori_loop(0, B_PER_W, _dot, 0)
    pltpu.sync_copy(outv, out_h.at[pl.ds(base, B_PER_W)])


_final = pl.kernel(
    _final_body,
    out_type=jax.ShapeDtypeStruct((BATCH,), jnp.float32),
    mesh=_mesh,
    compiler_params=pltpu.CompilerParams(
        use_tc_tiling_on_sc=False, needs_layout_passes=False),
    scratch_types=(
        [pltpu.VMEM((B_PER_W,), jnp.int32)] * 2
        + [pltpu.VMEM((B_PER_W, EMB), jnp.float32)] * 8
        + [pltpu.VMEM((B_PER_W,), jnp.float32)]
    ),
)


def kernel(user_embed, item_embed, adj_values, adj_indices, users, pos_items):
    all_embed = jnp.concatenate([user_embed, item_embed], axis=0)
    rows = adj_indices[0]
    cols = adj_indices[1]
    prow, pcol, pval = _partition(rows, cols, adj_values)
    e1 = _hop(all_embed, prow, pcol, pval)
    e2 = _hop(e1, prow, pcol, pval)
    e3 = _hop(e2, prow, pcol, pval)
    return _final(all_embed, e1, e2, e3, users, pos_items)


# speedup vs baseline: 3.9947x; 1.1085x over previous
import functools

import jax
import jax.numpy as jnp
from jax import lax
from jax.experimental import pallas as pl
from jax.experimental.pallas import tpu as pltpu
from jax.experimental.pallas import tpu_sc as plsc

N_USERS = 50000
N_TOTAL = 100000
EMB = 32
NNZ = 1600000
BATCH = 4096

NC = 2
NS = 16
L = 16

HALF = N_TOTAL // NC
ROWS_PER_TILE = 3128
ACC_ROWS = ROWS_PER_TILE * NS
LAST_ROWS = HALF - 15 * ROWS_PER_TILE

PSLICE = NNZ // (NC * NS)
SEG = 26000
CAPC = SEG * NC * NS
PCHUNK = 2000
PGROUPS = PCHUNK // L

EDGES_PER_TILE = CAPC // NS
E_CHUNK = 400
N_CHUNKS = EDGES_PER_TILE // E_CHUNK
GROUPS = E_CHUNK // L

SEGQ = 5600
CAPQ = SEGQ * NS
FLAG_ROWS = 50048
BSLICE = BATCH // NS

B_PER_W = BATCH // (NC * NS)

_mesh = plsc.VectorSubcoreMesh(core_axis_name="c", subcore_axis_name="s")


_GATHER_DNUMS = lax.GatherDimensionNumbers(
    offset_dims=(), collapsed_slice_dims=(0,), start_index_map=(0,))


def _bcast(vec, lane):
    idx = jnp.full((L, 1), lane, jnp.int32)
    return lax.gather(vec, idx, _GATHER_DNUMS, (1,),
                      mode=lax.GatherScatterMode.PROMISE_IN_BOUNDS)


def _partition_body(rows_h, cols_h, vals_h, prow_h, pcol_h, pval_h,
                    inr0, inr1, inc0, inc1, inv0, inv1, srow, scol, sval,
                    sem_p0, sem_p1):
    c = lax.axis_index("c")
    s = lax.axis_index("s")
    wid = s * NC + c
    ebase = wid * PSLICE
    inr = (inr0, inr1)
    inc = (inc0, inc1)
    inv = (inv0, inv1)
    sem_p = (sem_p0, sem_p1)

    izero = jnp.zeros((L,), jnp.int32)
    fzero = jnp.zeros((L,), jnp.float32)

    lane = lax.broadcasted_iota(jnp.int32, (L,), 0)

    def _pstage(k, b):
        eb = ebase + k * PCHUNK
        pltpu.async_copy(rows_h.at[pl.ds(eb, PCHUNK)], inr[b], sem_p[b])
        pltpu.async_copy(cols_h.at[pl.ds(eb, PCHUNK)], inc[b], sem_p[b])
        pltpu.async_copy(vals_h.at[pl.ds(eb, PCHUNK)], inv[b], sem_p[b])

    def _pstage_wait(k, b):
        eb = ebase + k * PCHUNK
        pltpu.make_async_copy(rows_h.at[pl.ds(eb, PCHUNK)], inr[b], sem_p[b]).wait()
        pltpu.make_async_copy(cols_h.at[pl.ds(eb, PCHUNK)], inc[b], sem_p[b]).wait()
        pltpu.make_async_copy(vals_h.at[pl.ds(eb, PCHUNK)], inv[b], sem_p[b]).wait()

    NPC = PSLICE // PCHUNK

    for r in range(NC):
        _pstage(0, 0)

        def _zb(g, _):
            pad = g * L + lane
            srow[pl.ds(g * L, L)] = pad
            scol[pl.ds(g * L, L)] = pad
            sval[pl.ds(g * L, L)] = fzero
            return 0

        lax.fori_loop(0, SEG // L, _zb, 0)

        def _outerp(gk, cur):
            for b in range(2):
                k = 2 * gk + b
                _pstage_wait(k, b)

                @pl.when(k + 1 < NPC)
                def _pf():
                    _pstage(k + 1, 1 - b)

                def _grp(g, cur):
                    r16 = inr[b][pl.ds(g * L, L)]
                    c16 = inc[b][pl.ds(g * L, L)]
                    v16 = inv[b][pl.ds(g * L, L)]
                    upper01 = 1 + ((r16 - HALF) >> 31)
                    folded = r16 - upper01 * HALF
                    mask = upper01 == r
                    n = plsc.all_reduce_population_count(mask)
                    plsc.store_compressed(srow.at[pl.ds(cur, L)], folded, mask=mask)
                    plsc.store_compressed(scol.at[pl.ds(cur, L)], c16, mask=mask)
                    plsc.store_compressed(sval.at[pl.ds(cur, L)], v16, mask=mask)
                    return jnp.minimum(cur + n[0], SEG)

                cur = lax.fori_loop(0, PGROUPS, _grp, cur)
            return cur

        cur_mid = lax.fori_loop(0, NPC // 2, _outerp, 0)

        _pstage_wait(NPC - 1, 0)

        def _grp_tail(g, cur):
            r16 = inr[0][pl.ds(g * L, L)]
            c16 = inc[0][pl.ds(g * L, L)]
            v16 = inv[0][pl.ds(g * L, L)]
            upper01 = 1 + ((r16 - HALF) >> 31)
            folded = r16 - upper01 * HALF
            mask = upper01 == r
            n = plsc.all_reduce_population_count(mask)
            plsc.store_compressed(srow.at[pl.ds(cur, L)], folded, mask=mask)
            plsc.store_compressed(scol.at[pl.ds(cur, L)], c16, mask=mask)
            plsc.store_compressed(sval.at[pl.ds(cur, L)], v16, mask=mask)
            return jnp.minimum(cur + n[0], SEG)

        cur_end = lax.fori_loop(0, PGROUPS, _grp_tail, cur_mid)
        srow[pl.ds(cur_end, L)] = cur_end + lane
        scol[pl.ds(cur_end, L)] = cur_end + lane
        sval[pl.ds(cur_end, L)] = fzero
        seg_base = r * CAPC + wid * SEG
        pltpu.sync_copy(srow.at[pl.ds(0, SEG)], prow_h.at[pl.ds(seg_base, SEG)])
        pltpu.sync_copy(scol.at[pl.ds(0, SEG)], pcol_h.at[pl.ds(seg_base, SEG)])
        pltpu.sync_copy(sval.at[pl.ds(0, SEG)], pval_h.at[pl.ds(seg_base, SEG)])


_partition = pl.kernel(
    _partition_body,
    out_type=(jax.ShapeDtypeStruct((NC * CAPC,), jnp.int32),
              jax.ShapeDtypeStruct((NC * CAPC,), jnp.int32),
              jax.ShapeDtypeStruct((NC * CAPC,), jnp.float32)),
    mesh=_mesh,
    compiler_params=pltpu.CompilerParams(
        use_tc_tiling_on_sc=False, needs_layout_passes=False),
    scratch_types=(
        [pltpu.VMEM((PCHUNK,), jnp.int32)] * 4
        + [pltpu.VMEM((PCHUNK,), jnp.float32)] * 2
        + [pltpu.VMEM((SEG + L,), jnp.int32)] * 2
        + [pltpu.VMEM((SEG + L,), jnp.float32)]
        + [pltpu.SemaphoreType.DMA] * 2
    ),
)


def _make_hop(cap, ept):
  n_chunks = ept // E_CHUNK

  def _hop_body(agg, rows_h, cols_h, vals_h, out_h,
              accum, colv0, colv1, rowv0, rowv1, valv0, valv1,
              idxv0, idxv1, gath0, gath1,
              sem_st0, sem_st1, sem_g0, sem_g1, sem_sc0, sem_sc1):
      c = lax.axis_index("c")
      s = lax.axis_index("s")
      colv = (colv0, colv1)
      rowv = (rowv0, rowv1)
      valv = (valv0, valv1)
      idxv = (idxv0, idxv1)
      gath = (gath0, gath1)
      sem_st = (sem_st0, sem_st1)
      sem_g = (sem_g0, sem_g1)
      sem_sc = (sem_sc0, sem_sc1)

      zero = jnp.zeros((L,), jnp.float32)

      def _zb(g, _):
          gath0[g, pl.ds(0, L)] = zero
          gath0[g, pl.ds(L, L)] = zero
          return 0

      lax.fori_loop(0, E_CHUNK, _zb, 0)
      zoff = 0
      while zoff < ROWS_PER_TILE:
          zn = min(E_CHUNK, ROWS_PER_TILE - zoff)
          pltpu.sync_copy(gath0.at[pl.ds(0, zn)],
                                accum.at[pl.ds(s * ROWS_PER_TILE + zoff, zn)])
          zoff += zn
      plsc.subcore_barrier()

      ebase = c * cap + s * ept

      def _stage(k, b):
          eb = ebase + k * E_CHUNK
          pltpu.async_copy(cols_h.at[pl.ds(eb, E_CHUNK)], colv[b], sem_st[b])
          pltpu.async_copy(rows_h.at[pl.ds(eb, E_CHUNK)], rowv[b], sem_st[b])
          pltpu.async_copy(vals_h.at[pl.ds(eb, E_CHUNK)], valv[b], sem_st[b])

      def _stage_wait(k, b):
          eb = ebase + k * E_CHUNK
          pltpu.make_async_copy(cols_h.at[pl.ds(eb, E_CHUNK)], colv[b], sem_st[b]).wait()
          pltpu.make_async_copy(rows_h.at[pl.ds(eb, E_CHUNK)], rowv[b], sem_st[b]).wait()
          pltpu.make_async_copy(vals_h.at[pl.ds(eb, E_CHUNK)], valv[b], sem_st[b]).wait()

      _stage(0, 0)
      _stage_wait(0, 0)
      pltpu.async_copy(agg.at[colv[0]], gath[0], sem_g[0])
      _stage(1, 1)

      def _outer(g, _):
          for b in range(2):
                    k = 2 * g + b
                    nb = 1 - b

                    @pl.when(k + 1 < n_chunks)
                    def _issue_next_gather():
                        _stage_wait(k + 1, nb)

                        @pl.when(k >= 1)
                        def _wait_prev_scatter():
                            pltpu.make_async_copy(
                                gath[nb], accum.at[idxv[nb]], sem_sc[nb]).wait()

                        pltpu.async_copy(agg.at[colv[nb]], gath[nb], sem_g[nb])

                    pltpu.make_async_copy(agg.at[colv[b]], gath[b], sem_g[b]).wait()

                    def _grp(gi, _):
                        v16 = valv[b][pl.ds(gi * L, L)]
                        idxv[b][pl.ds(gi * L, L)] = rowv[b][pl.ds(gi * L, L)]
                        e0 = gi * L
                        for e in range(L):
                            w = _bcast(v16, e)
                            gath[b][e0 + e, pl.ds(0, L)] = gath[b][e0 + e, pl.ds(0, L)] * w
                            gath[b][e0 + e, pl.ds(L, L)] = gath[b][e0 + e, pl.ds(L, L)] * w
                        return 0

                    lax.fori_loop(0, GROUPS, _grp, 0)

                    @pl.when(k + 2 < n_chunks)
                    def _prefetch():
                        _stage(k + 2, b)

                    pltpu.async_copy(gath[b], accum.at[idxv[b]], sem_sc[b], add=True)
          return 0

      lax.fori_loop(0, n_chunks // 2, _outer, 0)
      for b in range(2):
          pltpu.make_async_copy(gath[b], accum.at[idxv[b]], sem_sc[b]).wait()
      plsc.subcore_barrier()

      @pl.when(s < NS - 1)
      def _flush_full():
          pltpu.sync_copy(
                    accum.at[pl.ds(s * ROWS_PER_TILE, ROWS_PER_TILE)],
                    out_h.at[pl.ds(c * HALF + s * ROWS_PER_TILE, ROWS_PER_TILE)])

      @pl.when(s == NS - 1)
      def _flush_last():
          pltpu.sync_copy(
                    accum.at[pl.ds((NS - 1) * ROWS_PER_TILE, LAST_ROWS)],
                    out_h.at[pl.ds(c * HALF + (NS - 1) * ROWS_PER_TILE, LAST_ROWS)])


  return pl.kernel(
    _hop_body,
    out_type=jax.ShapeDtypeStruct((N_TOTAL, EMB), jnp.float32),
    mesh=_mesh,
    compiler_params=pltpu.CompilerParams(use_tc_tiling_on_sc=False),
    scratch_types=(
        [pltpu.VMEM_SHARED((ACC_ROWS, EMB), jnp.float32)]
        + [pltpu.VMEM((E_CHUNK,), jnp.int32)] * 4
        + [pltpu.VMEM((E_CHUNK,), jnp.float32)] * 2
        + [pltpu.VMEM((E_CHUNK,), jnp.int32)] * 2
        + [pltpu.VMEM((E_CHUNK, EMB), jnp.float32)] * 2
        + [pltpu.SemaphoreType.DMA] * 6
    ),
  )


_hop = _make_hop(CAPC, EDGES_PER_TILE)
_hop_pruned = _make_hop(CAPQ, SEGQ)


def _prune_body(prow_h, pcol_h, pval_h, users_h, pos_h,
                qrow_h, qcol_h, qval_h,
                flagv, bidx,
                inr0, inr1, inc0, inc1, inv0, inv1, srow, scol, sval,
                sem_p0, sem_p1):
    c = lax.axis_index("c")
    s = lax.axis_index("s")
    inr = (inr0, inr1)
    inc = (inc0, inc1)
    inv = (inv0, inv1)
    sem_p = (sem_p0, sem_p1)
    lane = lax.broadcasted_iota(jnp.int32, (L,), 0)
    izero = jnp.zeros((L,), jnp.int32)
    fzero = jnp.zeros((L,), jnp.float32)
    ione = jnp.full((L,), 1, jnp.int32)

    def _zf(g, _):
        flagv[pl.ds(g * L, L)] = izero
        return 0

    lax.fori_loop(0, FLAG_ROWS // L, _zf, 0)

    uval = ione * (1 - c)
    ival = ione * c

    pltpu.sync_copy(users_h, bidx)

    def _fsu(g, _):
        plsc.store_scatter(flagv, [bidx[pl.ds(g * L, L)]], uval)
        return 0

    lax.fori_loop(0, BATCH // L, _fsu, 0)
    pltpu.sync_copy(pos_h, bidx)

    def _fsi(g, _):
        plsc.store_scatter(flagv, [bidx[pl.ds(g * L, L)]], ival)
        return 0

    lax.fori_loop(0, BATCH // L, _fsi, 0)

    def _pad(g, _):
        pv = g * L + lane
        srow[pl.ds(g * L, L)] = pv
        scol[pl.ds(g * L, L)] = pv
        sval[pl.ds(g * L, L)] = fzero
        return 0

    lax.fori_loop(0, SEGQ // L, _pad, 0)

    ebase = c * CAPC + s * EDGES_PER_TILE
    NPQ = EDGES_PER_TILE // PCHUNK

    def _pstage(k, b):
        eb = ebase + k * PCHUNK
        pltpu.async_copy(prow_h.at[pl.ds(eb, PCHUNK)], inr[b], sem_p[b])
        pltpu.async_copy(pcol_h.at[pl.ds(eb, PCHUNK)], inc[b], sem_p[b])
        pltpu.async_copy(pval_h.at[pl.ds(eb, PCHUNK)], inv[b], sem_p[b])

    def _pstage_wait(k, b):
        eb = ebase + k * PCHUNK
        pltpu.make_async_copy(prow_h.at[pl.ds(eb, PCHUNK)], inr[b], sem_p[b]).wait()
        pltpu.make_async_copy(pcol_h.at[pl.ds(eb, PCHUNK)], inc[b], sem_p[b]).wait()
        pltpu.make_async_copy(pval_h.at[pl.ds(eb, PCHUNK)], inv[b], sem_p[b]).wait()

    _pstage(0, 0)

    def _outerq(gk, cur):
        for b in range(2):
            k = 2 * gk + b
            _pstage_wait(k, b)

            @pl.when(k + 1 < NPQ)
            def _pf():
                _pstage(k + 1, 1 - b)

            def _grp(g, cur):
                r16 = inr[b][pl.ds(g * L, L)]
                c16 = inc[b][pl.ds(g * L, L)]
                v16 = inv[b][pl.ds(g * L, L)]
                f16 = plsc.load_gather(flagv, [r16])
                mask = f16 > 0
                n = plsc.all_reduce_population_count(mask)
                plsc.store_compressed(srow.at[pl.ds(cur, L)], r16, mask=mask)
                plsc.store_compressed(scol.at[pl.ds(cur, L)], c16, mask=mask)
                plsc.store_compressed(sval.at[pl.ds(cur, L)], v16, mask=mask)
                return jnp.minimum(cur + n[0], SEGQ)

            cur = lax.fori_loop(0, PGROUPS, _grp, cur)
        return cur

    cur_end = lax.fori_loop(0, NPQ // 2, _outerq, 0)
    srow[pl.ds(cur_end, L)] = cur_end + lane
    scol[pl.ds(cur_end, L)] = cur_end + lane
    sval[pl.ds(cur_end, L)] = fzero

    qbase = c * CAPQ + s * SEGQ
    pltpu.sync_copy(srow.at[pl.ds(0, SEGQ)], qrow_h.at[pl.ds(qbase, SEGQ)])
    pltpu.sync_copy(scol.at[pl.ds(0, SEGQ)], qcol_h.at[pl.ds(qbase, SEGQ)])
    pltpu.sync_copy(sval.at[pl.ds(0, SEGQ)], qval_h.at[pl.ds(qbase, SEGQ)])


_prune = pl.kernel(
    _prune_body,
    out_type=(jax.ShapeDtypeStruct((NC * CAPQ,), jnp.int32),
              jax.ShapeDtypeStruct((NC * CAPQ,), jnp.int32),
              jax.ShapeDtypeStruct((NC * CAPQ,), jnp.float32)),
    mesh=_mesh,
    compiler_params=pltpu.CompilerParams(
        use_tc_tiling_on_sc=False, needs_layout_passes=False),
    scratch_types=(
        [pltpu.VMEM((FLAG_ROWS,), jnp.int32)]
        + [pltpu.VMEM((BATCH,), jnp.int32)]
        + [pltpu.VMEM((PCHUNK,), jnp.int32)] * 4
        + [pltpu.VMEM((PCHUNK,), jnp.float32)] * 2
        + [pltpu.VMEM((SEGQ + L,), jnp.int32)] * 2
        + [pltpu.VMEM((SEGQ + L,), jnp.float32)]
        + [pltpu.SemaphoreType.DMA] * 2
    ),
)


def _final_body(e0, e1, e2, e3, users_h, pos_h, out_h,
                uidx, pidx, ub0, ub1, ub2, ub3, ib0, ib1, ib2, ib3, outv):
    c = lax.axis_index("c")
    s = lax.axis_index("s")
    wid = s * NC + c
    base = wid * B_PER_W

    pltpu.sync_copy(users_h.at[pl.ds(base, B_PER_W)], uidx)
    pltpu.sync_copy(pos_h.at[pl.ds(base, B_PER_W)], pidx)

    def _shift(g, _):
        pidx[pl.ds(g * L, L)] = pidx[pl.ds(g * L, L)] + N_USERS
        return 0

    lax.fori_loop(0, B_PER_W // L, _shift, 0)

    pltpu.sync_copy(e0.at[uidx], ub0)
    pltpu.sync_copy(e1.at[uidx], ub1)
    pltpu.sync_copy(e2.at[uidx], ub2)
    pltpu.sync_copy(e3.at[uidx], ub3)
    pltpu.sync_copy(e0.at[pidx], ib0)
    pltpu.sync_copy(e1.at[pidx], ib1)
    pltpu.sync_copy(e2.at[pidx], ib2)
    pltpu.sync_copy(e3.at[pidx], ib3)

    def _dot(b, _):
        u0 = (ub0[b, pl.ds(0, L)] + ub1[b, pl.ds(0, L)]
              + ub2[b, pl.ds(0, L)] + ub3[b, pl.ds(0, L)])
        u1 = (ub0[b, pl.ds(L, L)] + ub1[b, pl.ds(L, L)]
              + ub2[b, pl.ds(L, L)] + ub3[b, pl.ds(L, L)])
        i0 = (ib0[b, pl.ds(0, L)] + ib1[b, pl.ds(0, L)]
              + ib2[b, pl.ds(0, L)] + ib3[b, pl.ds(0, L)])
        i1 = (ib0[b, pl.ds(L, L)] + ib1[b, pl.ds(L, L)]
              + ib2[b, pl.ds(L, L)] + ib3[b, pl.ds(L, L)])
        p = (u0 * i0 + u1 * i1) * (1.0 / 16.0)
        csum = plsc.cumsum(p)
        lane = lax.broadcasted_iota(jnp.int32, (L,), 0)
        plsc.store_scatter(outv, [jnp.full((L,), b, jnp.int32)], csum,
                           mask=lane == L - 1)
        return 0

    lax.fori_loop(0, B_PER_W, _dot, 0)
    pltpu.sync_copy(outv, out_h.at[pl.ds(base, B_PER_W)])


_final = pl.kernel(
    _final_body,
    out_type=jax.ShapeDtypeStruct((BATCH,), jnp.float32),
    mesh=_mesh,
    compiler_params=pltpu.CompilerParams(
        use_tc_tiling_on_sc=False, needs_layout_passes=False),
    scratch_types=(
        [pltpu.VMEM((B_PER_W,), jnp.int32)] * 2
        + [pltpu.VMEM((B_PER_W, EMB), jnp.float32)] * 8
        + [pltpu.VMEM((B_PER_W,), jnp.float32)]
    ),
)


def kernel(user_embed, item_embed, adj_values, adj_indices, users, pos_items):
    all_embed = jnp.concatenate([user_embed, item_embed], axis=0)
    rows = adj_indices[0]
    cols = adj_indices[1]
    prow, pcol, pval = _partition(rows, cols, adj_values)
    qrow, qcol, qval = _prune(prow, pcol, pval, users, pos_items)
    e1 = _hop(all_embed, prow, pcol, pval)
    e2 = _hop(e1, prow, pcol, pval)
    e3 = _hop_pruned(e2, qrow, qcol, qval)
    return _final(all_embed, e1, e2, e3, users, pos_items)


# speedup vs baseline: 3.9963x; 1.0004x over previous
import functools

import jax
import jax.numpy as jnp
from jax import lax
from jax.experimental import pallas as pl
from jax.experimental.pallas import tpu as pltpu
from jax.experimental.pallas import tpu_sc as plsc

N_USERS = 50000
N_TOTAL = 100000
EMB = 32
NNZ = 1600000
BATCH = 4096

NC = 2
NS = 16
L = 16

HALF = N_TOTAL // NC
ROWS_PER_TILE = 3128
ACC_ROWS = ROWS_PER_TILE * NS
LAST_ROWS = HALF - 15 * ROWS_PER_TILE

PSLICE = NNZ // (NC * NS)
SEG = 26000
CAPC = SEG * NC * NS
PCHUNK = 2000
PGROUPS = PCHUNK // L

EDGES_PER_TILE = CAPC // NS
E_CHUNK = 400
N_CHUNKS = EDGES_PER_TILE // E_CHUNK
GROUPS = E_CHUNK // L

SEGQ = 5600
CAPQ = SEGQ * NS
FLAG_ROWS = 50048
BSLICE = BATCH // NS

B_PER_W = BATCH // (NC * NS)

_mesh = plsc.VectorSubcoreMesh(core_axis_name="c", subcore_axis_name="s")


_GATHER_DNUMS = lax.GatherDimensionNumbers(
    offset_dims=(), collapsed_slice_dims=(0,), start_index_map=(0,))


def _bcast(vec, lane):
    idx = jnp.full((L, 1), lane, jnp.int32)
    return lax.gather(vec, idx, _GATHER_DNUMS, (1,),
                      mode=lax.GatherScatterMode.PROMISE_IN_BOUNDS)


def _partition_body(rows_h, cols_h, vals_h, prow_h, pcol_h, pval_h,
                    inr0, inr1, inc0, inc1, inv0, inv1, srow, scol, sval,
                    sem_p0, sem_p1):
    c = lax.axis_index("c")
    s = lax.axis_index("s")
    wid = s * NC + c
    ebase = wid * PSLICE
    inr = (inr0, inr1)
    inc = (inc0, inc1)
    inv = (inv0, inv1)
    sem_p = (sem_p0, sem_p1)

    izero = jnp.zeros((L,), jnp.int32)
    fzero = jnp.zeros((L,), jnp.float32)

    lane = lax.broadcasted_iota(jnp.int32, (L,), 0)

    def _pstage(k, b):
        eb = ebase + k * PCHUNK
        pltpu.async_copy(rows_h.at[pl.ds(eb, PCHUNK)], inr[b], sem_p[b])
        pltpu.async_copy(cols_h.at[pl.ds(eb, PCHUNK)], inc[b], sem_p[b])
        pltpu.async_copy(vals_h.at[pl.ds(eb, PCHUNK)], inv[b], sem_p[b])

    def _pstage_wait(k, b):
        eb = ebase + k * PCHUNK
        pltpu.make_async_copy(rows_h.at[pl.ds(eb, PCHUNK)], inr[b], sem_p[b]).wait()
        pltpu.make_async_copy(cols_h.at[pl.ds(eb, PCHUNK)], inc[b], sem_p[b]).wait()
        pltpu.make_async_copy(vals_h.at[pl.ds(eb, PCHUNK)], inv[b], sem_p[b]).wait()

    NPC = PSLICE // PCHUNK

    for r in range(NC):
        _pstage(0, 0)

        def _zb(g, _):
            pad = g * L + lane
            srow[pl.ds(g * L, L)] = pad
            scol[pl.ds(g * L, L)] = pad
            sval[pl.ds(g * L, L)] = fzero
            return 0

        lax.fori_loop(0, SEG // L, _zb, 0)

        def _outerp(gk, cur):
            for b in range(2):
                k = 2 * gk + b
                _pstage_wait(k, b)

                @pl.when(k + 1 < NPC)
                def _pf():
                    _pstage(k + 1, 1 - b)

                def _grp(g, cur):
                    r16 = inr[b][pl.ds(g * L, L)]
                    c16 = inc[b][pl.ds(g * L, L)]
                    v16 = inv[b][pl.ds(g * L, L)]
                    upper01 = 1 + ((r16 - HALF) >> 31)
                    folded = r16 - upper01 * HALF
                    mask = upper01 == r
                    n = plsc.all_reduce_population_count(mask)
                    plsc.store_compressed(srow.at[pl.ds(cur, L)], folded, mask=mask)
                    plsc.store_compressed(scol.at[pl.ds(cur, L)], c16, mask=mask)
                    plsc.store_compressed(sval.at[pl.ds(cur, L)], v16, mask=mask)
                    return jnp.minimum(cur + n[0], SEG)

                cur = lax.fori_loop(0, PGROUPS, _grp, cur)
            return cur

        cur_mid = lax.fori_loop(0, NPC // 2, _outerp, 0)

        _pstage_wait(NPC - 1, 0)

        def _grp_tail(g, cur):
            r16 = inr[0][pl.ds(g * L, L)]
            c16 = inc[0][pl.ds(g * L, L)]
            v16 = inv[0][pl.ds(g * L, L)]
            upper01 = 1 + ((r16 - HALF) >> 31)
            folded = r16 - upper01 * HALF
            mask = upper01 == r
            n = plsc.all_reduce_population_count(mask)
            plsc.store_compressed(srow.at[pl.ds(cur, L)], folded, mask=mask)
            plsc.store_compressed(scol.at[pl.ds(cur, L)], c16, mask=mask)
            plsc.store_compressed(sval.at[pl.ds(cur, L)], v16, mask=mask)
            return jnp.minimum(cur + n[0], SEG)

        cur_end = lax.fori_loop(0, PGROUPS, _grp_tail, cur_mid)
        srow[pl.ds(cur_end, L)] = cur_end + lane
        scol[pl.ds(cur_end, L)] = cur_end + lane
        sval[pl.ds(cur_end, L)] = fzero
        seg_base = r * CAPC + wid * SEG
        pltpu.sync_copy(srow.at[pl.ds(0, SEG)], prow_h.at[pl.ds(seg_base, SEG)])
        pltpu.sync_copy(scol.at[pl.ds(0, SEG)], pcol_h.at[pl.ds(seg_base, SEG)])
        pltpu.sync_copy(sval.at[pl.ds(0, SEG)], pval_h.at[pl.ds(seg_base, SEG)])


_partition = pl.kernel(
    _partition_body,
    out_type=(jax.ShapeDtypeStruct((NC * CAPC,), jnp.int32),
              jax.ShapeDtypeStruct((NC * CAPC,), jnp.int32),
              jax.ShapeDtypeStruct((NC * CAPC,), jnp.float32)),
    mesh=_mesh,
    compiler_params=pltpu.CompilerParams(
        use_tc_tiling_on_sc=False, needs_layout_passes=False),
    scratch_types=(
        [pltpu.VMEM((PCHUNK,), jnp.int32)] * 4
        + [pltpu.VMEM((PCHUNK,), jnp.float32)] * 2
        + [pltpu.VMEM((SEG + L,), jnp.int32)] * 2
        + [pltpu.VMEM((SEG + L,), jnp.float32)]
        + [pltpu.SemaphoreType.DMA] * 2
    ),
)


def _make_hop(cap, ept):
  n_chunks = ept // E_CHUNK

  def _hop_body(agg, rows_h, cols_h, vals_h, out_h,
              accum, colv0, colv1, rowv0, rowv1, valv0, valv1,
              idxv0, idxv1, gath0, gath1,
              sem_st0, sem_st1, sem_g0, sem_g1, sem_sc0, sem_sc1):
      c = lax.axis_index("c")
      s = lax.axis_index("s")
      colv = (colv0, colv1)
      rowv = (rowv0, rowv1)
      valv = (valv0, valv1)
      idxv = (idxv0, idxv1)
      gath = (gath0, gath1)
      sem_st = (sem_st0, sem_st1)
      sem_g = (sem_g0, sem_g1)
      sem_sc = (sem_sc0, sem_sc1)

      zero = jnp.zeros((L,), jnp.float32)

      def _zb(g, _):
          gath0[g, pl.ds(0, L)] = zero
          gath0[g, pl.ds(L, L)] = zero
          return 0

      lax.fori_loop(0, E_CHUNK, _zb, 0)
      zoff = 0
      while zoff < ROWS_PER_TILE:
          zn = min(E_CHUNK, ROWS_PER_TILE - zoff)
          pltpu.sync_copy(gath0.at[pl.ds(0, zn)],
                                accum.at[pl.ds(s * ROWS_PER_TILE + zoff, zn)])
          zoff += zn
      plsc.subcore_barrier()

      ebase = c * cap + s * ept

      def _stage(k, b):
          eb = ebase + k * E_CHUNK
          pltpu.async_copy(cols_h.at[pl.ds(eb, E_CHUNK)], colv[b], sem_st[b])
          pltpu.async_copy(rows_h.at[pl.ds(eb, E_CHUNK)], rowv[b], sem_st[b])
          pltpu.async_copy(vals_h.at[pl.ds(eb, E_CHUNK)], valv[b], sem_st[b])

      def _stage_wait(k, b):
          eb = ebase + k * E_CHUNK
          pltpu.make_async_copy(cols_h.at[pl.ds(eb, E_CHUNK)], colv[b], sem_st[b]).wait()
          pltpu.make_async_copy(rows_h.at[pl.ds(eb, E_CHUNK)], rowv[b], sem_st[b]).wait()
          pltpu.make_async_copy(vals_h.at[pl.ds(eb, E_CHUNK)], valv[b], sem_st[b]).wait()

      _stage(0, 0)
      _stage_wait(0, 0)
      pltpu.async_copy(agg.at[colv[0]], gath[0], sem_g[0])
      _stage(1, 1)

      def _outer(g, _):
          for b in range(2):
                    k = 2 * g + b
                    nb = 1 - b

                    @pl.when(k + 1 < n_chunks)
                    def _issue_next_gather():
                        _stage_wait(k + 1, nb)

                        @pl.when(k >= 1)
                        def _wait_prev_scatter():
                            pltpu.make_async_copy(
                                gath[nb], accum.at[idxv[nb]], sem_sc[nb]).wait()

                        pltpu.async_copy(agg.at[colv[nb]], gath[nb], sem_g[nb])

                    pltpu.make_async_copy(agg.at[colv[b]], gath[b], sem_g[b]).wait()

                    def _grp(gi, _):
                        v16 = valv[b][pl.ds(gi * L, L)]
                        idxv[b][pl.ds(gi * L, L)] = rowv[b][pl.ds(gi * L, L)]
                        e0 = gi * L
                        for e in range(L):
                            w = _bcast(v16, e)
                            gath[b][e0 + e, pl.ds(0, L)] = gath[b][e0 + e, pl.ds(0, L)] * w
                            gath[b][e0 + e, pl.ds(L, L)] = gath[b][e0 + e, pl.ds(L, L)] * w
                        return 0

                    lax.fori_loop(0, GROUPS, _grp, 0)

                    @pl.when(k + 2 < n_chunks)
                    def _prefetch():
                        _stage(k + 2, b)

                    pltpu.async_copy(gath[b], accum.at[idxv[b]], sem_sc[b], add=True)
          return 0

      lax.fori_loop(0, n_chunks // 2, _outer, 0)
      for b in range(2):
          pltpu.make_async_copy(gath[b], accum.at[idxv[b]], sem_sc[b]).wait()
      plsc.subcore_barrier()

      @pl.when(s < NS - 1)
      def _flush_full():
          pltpu.sync_copy(
                    accum.at[pl.ds(s * ROWS_PER_TILE, ROWS_PER_TILE)],
                    out_h.at[pl.ds(c * HALF + s * ROWS_PER_TILE, ROWS_PER_TILE)])

      @pl.when(s == NS - 1)
      def _flush_last():
          pltpu.sync_copy(
                    accum.at[pl.ds((NS - 1) * ROWS_PER_TILE, LAST_ROWS)],
                    out_h.at[pl.ds(c * HALF + (NS - 1) * ROWS_PER_TILE, LAST_ROWS)])


  return pl.kernel(
    _hop_body,
    out_type=jax.ShapeDtypeStruct((N_TOTAL, EMB), jnp.float32),
    mesh=_mesh,
    compiler_params=pltpu.CompilerParams(use_tc_tiling_on_sc=False),
    scratch_types=(
        [pltpu.VMEM_SHARED((ACC_ROWS, EMB), jnp.float32)]
        + [pltpu.VMEM((E_CHUNK,), jnp.int32)] * 4
        + [pltpu.VMEM((E_CHUNK,), jnp.float32)] * 2
        + [pltpu.VMEM((E_CHUNK,), jnp.int32)] * 2
        + [pltpu.VMEM((E_CHUNK, EMB), jnp.float32)] * 2
        + [pltpu.SemaphoreType.DMA] * 6
    ),
  )


_hop = _make_hop(CAPC, EDGES_PER_TILE)
_hop_pruned = _make_hop(CAPQ, SEGQ)


def _prune_body(prow_h, pcol_h, pval_h, users_h, pos_h,
                qrow_h, qcol_h, qval_h,
                flagv, bidx,
                inr0, inr1, inc0, inc1, inv0, inv1, srow, scol, sval,
                sem_p0, sem_p1):
    c = lax.axis_index("c")
    s = lax.axis_index("s")
    inr = (inr0, inr1)
    inc = (inc0, inc1)
    inv = (inv0, inv1)
    sem_p = (sem_p0, sem_p1)
    lane = lax.broadcasted_iota(jnp.int32, (L,), 0)
    izero = jnp.zeros((L,), jnp.int32)
    fzero = jnp.zeros((L,), jnp.float32)
    ione = jnp.full((L,), 1, jnp.int32)

    def _zf(g, _):
        flagv[pl.ds(g * L, L)] = izero
        return 0

    lax.fori_loop(0, FLAG_ROWS // L, _zf, 0)

    uval = ione * (1 - c)
    ival = ione * c

    pltpu.sync_copy(users_h, bidx)

    def _fsu(g, _):
        plsc.addupdate_scatter(flagv, [bidx[pl.ds(g * L, L)]], uval)
        return 0

    lax.fori_loop(0, BATCH // L, _fsu, 0)
    pltpu.sync_copy(pos_h, bidx)

    def _fsi(g, _):
        plsc.addupdate_scatter(flagv, [bidx[pl.ds(g * L, L)]], ival)
        return 0

    lax.fori_loop(0, BATCH // L, _fsi, 0)

    def _pad(g, _):
        pv = g * L + lane
        srow[pl.ds(g * L, L)] = pv
        scol[pl.ds(g * L, L)] = pv
        sval[pl.ds(g * L, L)] = fzero
        return 0

    lax.fori_loop(0, SEGQ // L, _pad, 0)

    ebase = c * CAPC + s * EDGES_PER_TILE
    NPQ = EDGES_PER_TILE // PCHUNK

    def _pstage(k, b):
        eb = ebase + k * PCHUNK
        pltpu.async_copy(prow_h.at[pl.ds(eb, PCHUNK)], inr[b], sem_p[b])
        pltpu.async_copy(pcol_h.at[pl.ds(eb, PCHUNK)], inc[b], sem_p[b])
        pltpu.async_copy(pval_h.at[pl.ds(eb, PCHUNK)], inv[b], sem_p[b])

    def _pstage_wait(k, b):
        eb = ebase + k * PCHUNK
        pltpu.make_async_copy(prow_h.at[pl.ds(eb, PCHUNK)], inr[b], sem_p[b]).wait()
        pltpu.make_async_copy(pcol_h.at[pl.ds(eb, PCHUNK)], inc[b], sem_p[b]).wait()
        pltpu.make_async_copy(pval_h.at[pl.ds(eb, PCHUNK)], inv[b], sem_p[b]).wait()

    _pstage(0, 0)

    def _outerq(gk, cur):
        for b in range(2):
            k = 2 * gk + b
            _pstage_wait(k, b)

            @pl.when(k + 1 < NPQ)
            def _pf():
                _pstage(k + 1, 1 - b)

            def _grp(g, cur):
                r16 = inr[b][pl.ds(g * L, L)]
                c16 = inc[b][pl.ds(g * L, L)]
                v16 = inv[b][pl.ds(g * L, L)]
                f16 = plsc.load_gather(flagv, [r16])
                mask = f16 > 0
                n = plsc.all_reduce_population_count(mask)
                plsc.store_compressed(srow.at[pl.ds(cur, L)], r16, mask=mask)
                plsc.store_compressed(scol.at[pl.ds(cur, L)], c16, mask=mask)
                plsc.store_compressed(sval.at[pl.ds(cur, L)], v16, mask=mask)
                return jnp.minimum(cur + n[0], SEGQ)

            cur = lax.fori_loop(0, PGROUPS, _grp, cur)
        return cur

    cur_end = lax.fori_loop(0, NPQ // 2, _outerq, 0)
    srow[pl.ds(cur_end, L)] = cur_end + lane
    scol[pl.ds(cur_end, L)] = cur_end + lane
    sval[pl.ds(cur_end, L)] = fzero

    qbase = c * CAPQ + s * SEGQ
    pltpu.sync_copy(srow.at[pl.ds(0, SEGQ)], qrow_h.at[pl.ds(qbase, SEGQ)])
    pltpu.sync_copy(scol.at[pl.ds(0, SEGQ)], qcol_h.at[pl.ds(qbase, SEGQ)])
    pltpu.sync_copy(sval.at[pl.ds(0, SEGQ)], qval_h.at[pl.ds(qbase, SEGQ)])


_prune = pl.kernel(
    _prune_body,
    out_type=(jax.ShapeDtypeStruct((NC * CAPQ,), jnp.int32),
              jax.ShapeDtypeStruct((NC * CAPQ,), jnp.int32),
              jax.ShapeDtypeStruct((NC * CAPQ,), jnp.float32)),
    mesh=_mesh,
    compiler_params=pltpu.CompilerParams(
        use_tc_tiling_on_sc=False, needs_layout_passes=False),
    scratch_types=(
        [pltpu.VMEM((FLAG_ROWS,), jnp.int32)]
        + [pltpu.VMEM((BATCH,), jnp.int32)]
        + [pltpu.VMEM((PCHUNK,), jnp.int32)] * 4
        + [pltpu.VMEM((PCHUNK,), jnp.float32)] * 2
        + [pltpu.VMEM((SEGQ + L,), jnp.int32)] * 2
        + [pltpu.VMEM((SEGQ + L,), jnp.float32)]
        + [pltpu.SemaphoreType.DMA] * 2
    ),
)


def _final_body(e0, e1, e2, e3, users_h, pos_h, out_h,
                uidx, pidx, ub0, ub1, ub2, ub3, ib0, ib1, ib2, ib3, outv):
    c = lax.axis_index("c")
    s = lax.axis_index("s")
    wid = s * NC + c
    base = wid * B_PER_W

    pltpu.sync_copy(users_h.at[pl.ds(base, B_PER_W)], uidx)
    pltpu.sync_copy(pos_h.at[pl.ds(base, B_PER_W)], pidx)

    def _shift(g, _):
        pidx[pl.ds(g * L, L)] = pidx[pl.ds(g * L, L)] + N_USERS
        return 0

    lax.fori_loop(0, B_PER_W // L, _shift, 0)

    pltpu.sync_copy(e0.at[uidx], ub0)
    pltpu.sync_copy(e1.at[uidx], ub1)
    pltpu.sync_copy(e2.at[uidx], ub2)
    pltpu.sync_copy(e3.at[uidx], ub3)
    pltpu.sync_copy(e0.at[pidx], ib0)
    pltpu.sync_copy(e1.at[pidx], ib1)
    pltpu.sync_copy(e2.at[pidx], ib2)
    pltpu.sync_copy(e3.at[pidx], ib3)

    def _dot(b, _):
        u0 = (ub0[b, pl.ds(0, L)] + ub1[b, pl.ds(0, L)]
              + ub2[b, pl.ds(0, L)] + ub3[b, pl.ds(0, L)])
        u1 = (ub0[b, pl.ds(L, L)] + ub1[b, pl.ds(L, L)]
              + ub2[b, pl.ds(L, L)] + ub3[b, pl.ds(L, L)])
        i0 = (ib0[b, pl.ds(0, L)] + ib1[b, pl.ds(0, L)]
              + ib2[b, pl.ds(0, L)] + ib3[b, pl.ds(0, L)])
        i1 = (ib0[b, pl.ds(L, L)] + ib1[b, pl.ds(L, L)]
              + ib2[b, pl.ds(L, L)] + ib3[b, pl.ds(L, L)])
        p = (u0 * i0 + u1 * i1) * (1.0 / 16.0)
        csum = plsc.cumsum(p)
        lane = lax.broadcasted_iota(jnp.int32, (L,), 0)
        plsc.store_scatter(outv, [jnp.full((L,), b, jnp.int32)], csum,
                           mask=lane == L - 1)
        return 0

    lax.fori_loop(0, B_PER_W, _dot, 0)
    pltpu.sync_copy(outv, out_h.at[pl.ds(base, B_PER_W)])


_final = pl.kernel(
    _final_body,
    out_type=jax.ShapeDtypeStruct((BATCH,), jnp.float32),
    mesh=_mesh,
    compiler_params=pltpu.CompilerParams(
        use_tc_tiling_on_sc=False, needs_layout_passes=False),
    scratch_types=(
        [pltpu.VMEM((B_PER_W,), jnp.int32)] * 2
        + [pltpu.VMEM((B_PER_W, EMB), jnp.float32)] * 8
        + [pltpu.VMEM((B_PER_W,), jnp.float32)]
    ),
)


def kernel(user_embed, item_embed, adj_values, adj_indices, users, pos_items):
    all_embed = jnp.concatenate([user_embed, item_embed], axis=0)
    rows = adj_indices[0]
    cols = adj_indices[1]
    prow, pcol, pval = _partition(rows, cols, adj_values)
    qrow, qcol, qval = _prune(prow, pcol, pval, users, pos_items)
    e1 = _hop(all_embed, prow, pcol, pval)
    e2 = _hop(e1, prow, pcol, pval)
    e3 = _hop_pruned(e2, qrow, qcol, qval)
    return _final(all_embed, e1, e2, e3, users, pos_items)
